# baseline (speedup 1.0000x reference)
import functools

import jax
import jax.numpy as jnp
from jax.experimental import pallas as pl

N = 10000
H = 128
A = 64
R1 = 400
R2 = 2000
RP = 1000
BN_EPS = 1e-5
ASCALE = 2.0 ** 13
DESCALE = 2.0 ** -13


def _row_spec(r, c):
    return pl.BlockSpec((r, c), lambda i: (i, 0))


def _const_spec(*shape):
    return pl.BlockSpec(shape, lambda i: (0,) * len(shape))


def _prep_kernel(x_ref, w_ref, z_ref):
    z_ref[...] = jnp.dot(
        x_ref[...].astype(jnp.bfloat16), w_ref[...].astype(jnp.bfloat16),
        preferred_element_type=jnp.float32).astype(jnp.bfloat16)


def _l1_hot_kernel(adj_ref, z_ref, adjq_ref, acc_ref):
    a = adj_ref[...]
    adjq_ref[...] = (a * ASCALE).astype(jnp.float8_e4m3fn)
    acc_ref[...] = jnp.dot(
        a.astype(jnp.bfloat16), z_ref[...],
        preferred_element_type=jnp.float32).astype(jnp.bfloat16)


def _l23_hot_kernel(adjq_ref, zq_ref, acc_ref):
    acc_ref[...] = jnp.dot(
        adjq_ref[...], zq_ref[...],
        preferred_element_type=jnp.float32).astype(jnp.bfloat16)


def _bn_elu_res(acc, idv, g, be, b, descale):
    cb = 1.0 / jnp.sqrt(jnp.float32(1.0 + BN_EPS))
    sg = g * (descale * cb)
    bias = b * (g * cb) + be
    h = acc * sg + bias
    h = jnp.where(h > 0, h, jnp.exp(jnp.minimum(h, 0.0)) - 1.0)
    return h + idv


def _ep_attn_kernel(acc_ref, id_ref, g_ref, be_ref, b_ref,
                    aW1_ref, ab1_ref, aW2_ref, ab2_ref, wn_ref,
                    h_ref, zq_ref, *, descale):
    acc = acc_ref[...].astype(jnp.float32)
    h = _bn_elu_res(acc, id_ref[...], g_ref[...], be_ref[...], b_ref[...],
                    descale)
    a = jnp.maximum(
        jnp.dot(h, aW1_ref[...], preferred_element_type=jnp.float32)
        + ab1_ref[...], 0.0)
    logit = jnp.sum(a * aW2_ref[...], axis=1, keepdims=True) + ab2_ref[...]
    h = h * jax.nn.sigmoid(logit)
    h_ref[...] = h
    zq_ref[...] = jnp.dot(
        h.astype(jnp.bfloat16), wn_ref[...].astype(jnp.bfloat16),
        preferred_element_type=jnp.float32).astype(jnp.float8_e4m3fn)


def _ep_fc_kernel(acc_ref, id_ref, g_ref, be_ref, b_ref,
                  fcW_ref, fcb_ref, out_ref):
    acc = acc_ref[...].astype(jnp.float32)
    h = _bn_elu_res(acc, id_ref[...], g_ref[...], be_ref[...], b_ref[...],
                    DESCALE)
    out_ref[...] = (
        jnp.dot(h, fcW_ref[...], preferred_element_type=jnp.float32)
        + fcb_ref[...])


def _ep_attn(acc, ident, g, be, b, aW1, ab1, aW2, ab2, wn, *, descale):
    return pl.pallas_call(
        functools.partial(_ep_attn_kernel, descale=descale),
        grid=(N // RP,),
        in_specs=[_row_spec(RP, H), _row_spec(RP, H), _const_spec(1, H),
                  _const_spec(1, H), _const_spec(1, H), _const_spec(H, A),
                  _const_spec(1, A), _const_spec(1, A), _const_spec(1, 1),
                  _const_spec(H, H)],
        out_specs=[_row_spec(RP, H), _row_spec(RP, H)],
        out_shape=[
            jax.ShapeDtypeStruct((N, H), jnp.float32),
            jax.ShapeDtypeStruct((N, H), jnp.float8_e4m3fn),
        ],
    )(acc, ident, g.reshape(1, H), be.reshape(1, H), b.reshape(1, H),
      aW1, ab1.reshape(1, A), aW2.reshape(1, A), ab2.reshape(1, 1), wn)


def kernel(x, adj, W1, b1, W2, b2, W3, b3, g1, be1, g2, be2, g3, be3,
           a1W1, a1b1, a1W2, a1b2, a2W1, a2b1, a2W2, a2b2, fcW, fcb):
    z1 = pl.pallas_call(
        _prep_kernel,
        grid=(N // RP,),
        in_specs=[_row_spec(RP, H), _const_spec(H, H)],
        out_specs=_row_spec(RP, H),
        out_shape=jax.ShapeDtypeStruct((N, H), jnp.bfloat16),
    )(x, W1)

    adjq, acc1 = pl.pallas_call(
        _l1_hot_kernel,
        grid=(N // R1,),
        in_specs=[_row_spec(R1, N), _const_spec(N, H)],
        out_specs=[_row_spec(R1, N), _row_spec(R1, H)],
        out_shape=[
            jax.ShapeDtypeStruct((N, N), jnp.float8_e4m3fn),
            jax.ShapeDtypeStruct((N, H), jnp.bfloat16),
        ],
    )(adj, z1)

    h1, zq2 = _ep_attn(acc1, x, g1, be1, b1,
                       a1W1, a1b1, a1W2, a1b2, W2, descale=1.0)

    def hot23(zq):
        return pl.pallas_call(
            _l23_hot_kernel,
            grid=(N // R2,),
            in_specs=[_row_spec(R2, N), _const_spec(N, H)],
            out_specs=_row_spec(R2, H),
            out_shape=jax.ShapeDtypeStruct((N, H), jnp.bfloat16),
        )(adjq, zq)

    acc2 = hot23(zq2)
    h2, zq3 = _ep_attn(acc2, h1, g2, be2, b2,
                       a2W1, a2b1, a2W2, a2b2, W3, descale=DESCALE)

    acc3 = hot23(zq3)
    out = pl.pallas_call(
        _ep_fc_kernel,
        grid=(N // RP,),
        in_specs=[_row_spec(RP, H), _row_spec(RP, H), _const_spec(1, H),
                  _const_spec(1, H), _const_spec(1, H),
                  _const_spec(H, H), _const_spec(1, H)],
        out_specs=_row_spec(RP, H),
        out_shape=jax.ShapeDtypeStruct((N, H), jnp.float32),
    )(acc3, h2, g3.reshape(1, H), be3.reshape(1, H),
      b3.reshape(1, H), fcW, fcb.reshape(1, H))

    return out


# speedup vs baseline: 1.0546x; 1.0546x over previous
import functools

import jax
import jax.numpy as jnp
from jax.experimental import pallas as pl

N = 10000
H = 128
A = 64
R1 = 400
R2 = 1000
RP = 2000
BN_EPS = 1e-5
ASCALE = 2.0 ** 13
DESCALE = 2.0 ** -13


def _row_spec(r, c):
    return pl.BlockSpec((r, c), lambda i: (i, 0))


def _const_spec(*shape):
    return pl.BlockSpec(shape, lambda i: (0,) * len(shape))


def _prep_kernel(x_ref, w_ref, z_ref):
    z_ref[...] = jnp.dot(
        x_ref[...].astype(jnp.bfloat16), w_ref[...].astype(jnp.bfloat16),
        preferred_element_type=jnp.float32).astype(jnp.bfloat16)


def _l1_hot_kernel(adj_ref, z_ref, adjq_ref, acc_ref):
    a = adj_ref[...]
    adjq_ref[...] = (a * ASCALE).astype(jnp.float8_e4m3fn)
    acc_ref[...] = jnp.dot(
        a.astype(jnp.bfloat16), z_ref[...],
        preferred_element_type=jnp.float32).astype(jnp.bfloat16)


def _l23_hot_kernel(adjq_ref, zq_ref, acc_ref):
    acc_ref[...] = jnp.dot(
        adjq_ref[...], zq_ref[...],
        preferred_element_type=jnp.float32).astype(jnp.bfloat16)


def _bn_elu_res(acc, idv, g, be, b, descale):
    cb = 1.0 / jnp.sqrt(jnp.float32(1.0 + BN_EPS))
    sg = g * (descale * cb)
    bias = b * (g * cb) + be
    h = acc * sg + bias
    h = jnp.where(h > 0, h, jnp.exp(jnp.minimum(h, 0.0)) - 1.0)
    return h + idv


def _ep_attn_kernel(acc_ref, id_ref, g_ref, be_ref, b_ref,
                    aW1_ref, ab1_ref, aW2_ref, ab2_ref, wn_ref,
                    h_ref, zq_ref, *, descale):
    acc = acc_ref[...].astype(jnp.float32)
    h = _bn_elu_res(acc, id_ref[...], g_ref[...], be_ref[...], b_ref[...],
                    descale)
    a = jnp.maximum(
        jnp.dot(h, aW1_ref[...], preferred_element_type=jnp.float32)
        + ab1_ref[...], 0.0)
    logit = jnp.sum(a * aW2_ref[...], axis=1, keepdims=True) + ab2_ref[...]
    h = h * jax.nn.sigmoid(logit)
    h_ref[...] = h
    zq_ref[...] = jnp.dot(
        h.astype(jnp.bfloat16), wn_ref[...].astype(jnp.bfloat16),
        preferred_element_type=jnp.float32).astype(jnp.float8_e4m3fn)


def _ep_fc_kernel(acc_ref, id_ref, g_ref, be_ref, b_ref,
                  fcW_ref, fcb_ref, out_ref):
    acc = acc_ref[...].astype(jnp.float32)
    h = _bn_elu_res(acc, id_ref[...], g_ref[...], be_ref[...], b_ref[...],
                    DESCALE)
    out_ref[...] = (
        jnp.dot(h, fcW_ref[...], preferred_element_type=jnp.float32)
        + fcb_ref[...])


def _ep_attn(acc, ident, g, be, b, aW1, ab1, aW2, ab2, wn, *, descale):
    return pl.pallas_call(
        functools.partial(_ep_attn_kernel, descale=descale),
        grid=(N // RP,),
        in_specs=[_row_spec(RP, H), _row_spec(RP, H), _const_spec(1, H),
                  _const_spec(1, H), _const_spec(1, H), _const_spec(H, A),
                  _const_spec(1, A), _const_spec(1, A), _const_spec(1, 1),
                  _const_spec(H, H)],
        out_specs=[_row_spec(RP, H), _row_spec(RP, H)],
        out_shape=[
            jax.ShapeDtypeStruct((N, H), jnp.float32),
            jax.ShapeDtypeStruct((N, H), jnp.float8_e4m3fn),
        ],
    )(acc, ident, g.reshape(1, H), be.reshape(1, H), b.reshape(1, H),
      aW1, ab1.reshape(1, A), aW2.reshape(1, A), ab2.reshape(1, 1), wn)


def kernel(x, adj, W1, b1, W2, b2, W3, b3, g1, be1, g2, be2, g3, be3,
           a1W1, a1b1, a1W2, a1b2, a2W1, a2b1, a2W2, a2b2, fcW, fcb):
    z1 = pl.pallas_call(
        _prep_kernel,
        grid=(N // RP,),
        in_specs=[_row_spec(RP, H), _const_spec(H, H)],
        out_specs=_row_spec(RP, H),
        out_shape=jax.ShapeDtypeStruct((N, H), jnp.bfloat16),
    )(x, W1)

    adjq, acc1 = pl.pallas_call(
        _l1_hot_kernel,
        grid=(N // R1,),
        in_specs=[_row_spec(R1, N), _const_spec(N, H)],
        out_specs=[_row_spec(R1, N), _row_spec(R1, H)],
        out_shape=[
            jax.ShapeDtypeStruct((N, N), jnp.float8_e4m3fn),
            jax.ShapeDtypeStruct((N, H), jnp.bfloat16),
        ],
    )(adj, z1)

    h1, zq2 = _ep_attn(acc1, x, g1, be1, b1,
                       a1W1, a1b1, a1W2, a1b2, W2, descale=1.0)

    def hot23(zq):
        return pl.pallas_call(
            _l23_hot_kernel,
            grid=(N // R2,),
            in_specs=[_row_spec(R2, N), _const_spec(N, H)],
            out_specs=_row_spec(R2, H),
            out_shape=jax.ShapeDtypeStruct((N, H), jnp.bfloat16),
        )(adjq, zq)

    acc2 = hot23(zq2)
    h2, zq3 = _ep_attn(acc2, h1, g2, be2, b2,
                       a2W1, a2b1, a2W2, a2b2, W3, descale=DESCALE)

    acc3 = hot23(zq3)
    out = pl.pallas_call(
        _ep_fc_kernel,
        grid=(N // RP,),
        in_specs=[_row_spec(RP, H), _row_spec(RP, H), _const_spec(1, H),
                  _const_spec(1, H), _const_spec(1, H),
                  _const_spec(H, H), _const_spec(1, H)],
        out_specs=_row_spec(RP, H),
        out_shape=jax.ShapeDtypeStruct((N, H), jnp.float32),
    )(acc3, h2, g3.reshape(1, H), be3.reshape(1, H),
      b3.reshape(1, H), fcW, fcb.reshape(1, H))

    return out
